# baseline (speedup 1.0000x reference)
import functools

import jax
import jax.numpy as jnp
from jax import lax
from jax.experimental import pallas as pl
from jax.experimental.pallas import tpu as pltpu
from jax.experimental.pallas import tpu_sc as plsc

FEATS = 512
ROWS = 16
NTIMES = 16384
NC, NS, L = 2, 16, 16
NW = NC * NS
CPW = NTIMES // NW
NGROUPS = CPW // L
EBLK = 32
NBLK = CPW // EBLK
NCH = FEATS // L

_mesh = plsc.VectorSubcoreMesh(core_axis_name="c", subcore_axis_name="s")


@functools.partial(
    pl.kernel,
    mesh=_mesh,
    compiler_params=pltpu.CompilerParams(needs_layout_passes=False),
    out_type=jax.ShapeDtypeStruct((NTIMES, FEATS), jnp.float32),
    scratch_types=[
        pltpu.VMEM((CPW,), jnp.float32),
        pltpu.VMEM((ROWS * FEATS,), jnp.float32),
        pltpu.VMEM((CPW + L,), jnp.int32),
        pltpu.VMEM((CPW + L,), jnp.int32),
        pltpu.VMEM((CPW + L,), jnp.float32),
        pltpu.VMEM((EBLK, FEATS), jnp.float32),
        pltpu.VMEM((EBLK, FEATS), jnp.float32),
        pltpu.SemaphoreType.DMA,
        pltpu.SemaphoreType.DMA,
    ],
)
def _sc_interp(times_hbm, table_hbm, out_hbm,
               times_v, table_v, lb_v, rb_v, lw_v,
               out0_v, out1_v, sem0, sem1):
    wid = lax.axis_index("s") * NC + lax.axis_index("c")
    base = wid * CPW
    pltpu.sync_copy(times_hbm.at[pl.ds(base, CPW)], times_v)
    pltpu.sync_copy(table_hbm, table_v)

    @plsc.parallel_loop(0, NGROUPS, unroll=4)
    def weight_body(g):
        t = times_v[pl.ds(g * L, L)]
        data = t * float(ROWS)
        li = jnp.clip(data, 0.0, float(ROWS - 1)).astype(jnp.int32)
        ri = jnp.minimum(li + 1, ROWS - 1)
        lw = data - li.astype(jnp.float32)
        sl = pl.ds(g * L, L)
        lb_v[sl] = li * FEATS
        rb_v[sl] = ri * FEATS
        lw_v[sl] = lw

    iota_c = lax.iota(jnp.int32, L)

    def compute_block(b, buf):
        eb = b * EBLK

        @plsc.parallel_loop(0, EBLK, unroll=2)
        def elem_body(e):
            eidx = jnp.full((L,), eb + e, dtype=jnp.int32)
            lbv = plsc.load_gather(lb_v, [eidx]) + iota_c
            rbv = plsc.load_gather(rb_v, [eidx]) + iota_c
            lwv = plsc.load_gather(lw_v, [eidx])
            rwv = 1.0 - lwv
            for k in range(NCH):
                le = plsc.load_gather(table_v, [lbv + (k * L)])
                re = plsc.load_gather(table_v, [rbv + (k * L)])
                buf[e, pl.ds(k * L, L)] = rwv * le + lwv * re

    def out_slice(b):
        return out_hbm.at[pl.ds(base + b * EBLK, EBLK)]

    compute_block(0, out0_v)
    pltpu.async_copy(out0_v, out_slice(0), sem0)
    compute_block(1, out1_v)
    pltpu.async_copy(out1_v, out_slice(1), sem1)

    def pair_body(p, carry):
        b0 = 2 * p
        pltpu.make_async_copy(out0_v, out_slice(b0), sem0).wait()
        compute_block(b0, out0_v)
        pltpu.async_copy(out0_v, out_slice(b0), sem0)
        b1 = 2 * p + 1
        pltpu.make_async_copy(out1_v, out_slice(b1), sem1).wait()
        compute_block(b1, out1_v)
        pltpu.async_copy(out1_v, out_slice(b1), sem1)
        return carry

    lax.fori_loop(1, NBLK // 2, pair_body, 0)

    pltpu.make_async_copy(out0_v, out_slice(0), sem0).wait()
    pltpu.make_async_copy(out1_v, out_slice(1), sem1).wait()


def kernel(times, table):
    return _sc_interp(times, table.reshape(ROWS * FEATS))


# speedup vs baseline: 1.9125x; 1.9125x over previous
import functools

import jax
import jax.numpy as jnp
from jax import lax
from jax.experimental import pallas as pl
from jax.experimental.pallas import tpu as pltpu
from jax.experimental.pallas import tpu_sc as plsc

FEATS = 512
ROWS = 16
NTIMES = 16384
NC, NS, L = 2, 16, 16
NW = NC * NS
CPW = NTIMES // NW
NGROUPS = CPW // L
EBLK = 16
NBLK = CPW // EBLK
NCH = FEATS // L

_mesh = plsc.VectorSubcoreMesh(core_axis_name="c", subcore_axis_name="s")


@functools.partial(
    pl.kernel,
    mesh=_mesh,
    compiler_params=pltpu.CompilerParams(needs_layout_passes=False),
    out_type=jax.ShapeDtypeStruct((NTIMES, FEATS), jnp.float32),
    scratch_types=[
        pltpu.VMEM((CPW,), jnp.float32),
        pltpu.VMEM((ROWS * FEATS,), jnp.float32),
        pltpu.VMEM((CPW + L,), jnp.int32),
        pltpu.VMEM((CPW + L,), jnp.int32),
        pltpu.VMEM((CPW + L,), jnp.float32),
        pltpu.VMEM((EBLK, FEATS), jnp.float32),
        pltpu.VMEM((EBLK, FEATS), jnp.float32),
        pltpu.SemaphoreType.DMA,
        pltpu.SemaphoreType.DMA,
    ],
)
def _sc_interp(times_hbm, table_hbm, out_hbm,
               times_v, table_v, lb_v, rb_v, lw_v,
               out0_v, out1_v, sem0, sem1):
    wid = lax.axis_index("s") * NC + lax.axis_index("c")
    base = wid * CPW
    pltpu.sync_copy(times_hbm.at[pl.ds(base, CPW)], times_v)
    pltpu.sync_copy(table_hbm, table_v)

    @plsc.parallel_loop(0, NGROUPS, unroll=4)
    def weight_body(g):
        t = times_v[pl.ds(g * L, L)]
        data = t * float(ROWS)
        li = jnp.clip(data, 0.0, float(ROWS - 1)).astype(jnp.int32)
        ri = jnp.minimum(li + 1, ROWS - 1)
        lw = data - li.astype(jnp.float32)
        sl = pl.ds(g * L, L)
        lb_v[sl] = li * FEATS
        rb_v[sl] = ri * FEATS
        lw_v[sl] = lw

    iota_c = lax.iota(jnp.int32, L)

    def compute_block(b, buf):
        eb = b * EBLK

        @plsc.parallel_loop(0, EBLK, unroll=4)
        def elem_body(e):
            eidx = jnp.full((L,), eb + e, dtype=jnp.int32)
            lbv = plsc.load_gather(lb_v, [eidx]) + iota_c
            rbv = plsc.load_gather(rb_v, [eidx]) + iota_c
            lwv = plsc.load_gather(lw_v, [eidx])
            rwv = 1.0 - lwv
            for k in range(NCH):
                le = plsc.load_gather(table_v, [lbv + (k * L)])
                re = plsc.load_gather(table_v, [rbv + (k * L)])
                buf[e, pl.ds(k * L, L)] = rwv * le + lwv * re

    def out_slice(b):
        return out_hbm.at[pl.ds(base + b * EBLK, EBLK)]

    compute_block(0, out0_v)
    pltpu.async_copy(out0_v, out_slice(0), sem0)
    compute_block(1, out1_v)
    pltpu.async_copy(out1_v, out_slice(1), sem1)

    def pair_body(p, carry):
        b0 = 2 * p
        pltpu.make_async_copy(out0_v, out_slice(b0), sem0).wait()
        compute_block(b0, out0_v)
        pltpu.async_copy(out0_v, out_slice(b0), sem0)
        b1 = 2 * p + 1
        pltpu.make_async_copy(out1_v, out_slice(b1), sem1).wait()
        compute_block(b1, out1_v)
        pltpu.async_copy(out1_v, out_slice(b1), sem1)
        return carry

    lax.fori_loop(1, NBLK // 2, pair_body, 0)

    pltpu.make_async_copy(out0_v, out_slice(0), sem0).wait()
    pltpu.make_async_copy(out1_v, out_slice(1), sem1).wait()


def kernel(times, table):
    return _sc_interp(times, table.reshape(ROWS * FEATS))
